# baseline (speedup 1.0000x reference)
import functools

import jax
import jax.numpy as jnp
from jax import lax
from jax.experimental import pallas as pl
from jax.experimental.pallas import tpu as pltpu
from jax.experimental.pallas import tpu_sc as plsc

B, P, S, K = 16, 2048, 256, 32
RADIUS = 0.15
R2 = RADIUS * RADIUS
NEG_INF = float("-inf")


def _fps_body(pxyz_ref, sel_ref, fxyz_ref):
    pxyz = pxyz_ref[...]
    px = pxyz[0:B]
    py = pxyz[B:2 * B]
    pz = pxyz[2 * B:3 * B]
    iota_p3 = lax.broadcasted_iota(jnp.int32, (3 * B, P), 1)
    iota_s = lax.broadcasted_iota(jnp.int32, (B, S), 1)
    iota_s3 = lax.broadcasted_iota(jnp.int32, (3 * B, S), 1)

    l0 = pxyz[:, 0:1]
    sel0 = jnp.zeros((B, S), jnp.int32)
    fxyz0 = jnp.where(iota_s3 == 0, l0, 0.0)
    dist0 = jnp.full((B, P), jnp.inf, jnp.float32)

    def body(i, carry):
        dist, sel, fxyz, l = carry
        lx = l[0:B]
        ly = l[B:2 * B]
        lz = l[2 * B:3 * B]
        dx = px - lx
        dy = py - ly
        dz = pz - lz
        d = (dx * dx + dy * dy) + dz * dz
        dist = jnp.minimum(dist, d)
        nxt = jnp.argmax(dist, axis=1).astype(jnp.int32)[:, None]
        nxt3 = jnp.concatenate([nxt, nxt, nxt], axis=0)
        onehot = iota_p3 == nxt3
        l = jnp.sum(jnp.where(onehot, pxyz, 0.0), axis=1, keepdims=True)
        sel = jnp.where(iota_s == i, nxt, sel)
        fxyz = jnp.where(iota_s3 == i, l, fxyz)
        return (dist, sel, fxyz, l)

    carry = (dist0, sel0, fxyz0, l0)
    _, sel, fxyz, _ = lax.fori_loop(1, S, body, carry)
    sel_ref[...] = sel
    fxyz_ref[...] = fxyz


def _run_fps(pxyz):
    return pl.pallas_call(
        _fps_body,
        out_shape=(
            jax.ShapeDtypeStruct((B, S), jnp.int32),
            jax.ShapeDtypeStruct((3 * B, S), jnp.float32),
        ),
    )(pxyz)


NWORK = 32
CPW = (B * S) // NWORK
CAND = 2064


def _sc_body(pxyz, batch, fxyz, sel,
             rel4_o, fpsb_o,
             px_v, py_v, pz_v, bat_v, cx_v, cy_v, cz_v, sel_v,
             cd2_v, cix_v, pick_v, rel4_v, fpsb_v):
    wid = lax.axis_index("s") * 2 + lax.axis_index("c")
    b = wid // 2
    half = wid % 2

    pltpu.sync_copy(pxyz.at[b], px_v)
    pltpu.sync_copy(pxyz.at[B + b], py_v)
    pltpu.sync_copy(pxyz.at[2 * B + b], pz_v)
    pltpu.sync_copy(batch.at[pl.ds(b * P, P)], bat_v)
    pltpu.sync_copy(fxyz.at[b, pl.ds(half * CPW, CPW)], cx_v)
    pltpu.sync_copy(fxyz.at[B + b, pl.ds(half * CPW, CPW)], cy_v)
    pltpu.sync_copy(fxyz.at[2 * B + b, pl.ds(half * CPW, CPW)], cz_v)
    pltpu.sync_copy(sel.at[b, pl.ds(half * CPW, CPW)], sel_v)

    lane = lax.iota(jnp.int32, 16)

    zero16f = jnp.zeros((16,), jnp.float32)

    @plsc.parallel_loop(0, CPW * K * 4 // 16)
    def _zero_body(j):
        rel4_v[pl.ds(j * 16, 16)] = zero16f

    for j in range(CPW // 16):
        sv = sel_v[pl.ds(j * 16, 16)]
        fpsb_v[pl.ds(j * 16, 16)] = plsc.load_gather(bat_v, [sv])

    inf16 = jnp.full((16,), jnp.inf, jnp.float32)

    def center_body(i, _):
        spl_i = jnp.full((16,), 0, jnp.int32) + i
        cxs = plsc.load_gather(cx_v, [spl_i])
        cys = plsc.load_gather(cy_v, [spl_i])
        czs = plsc.load_gather(cz_v, [spl_i])
        ctr_idx = plsc.load_gather(sel_v, [spl_i])

        for j in range(8):
            cd2_v[pl.ds(j * 16, 16)] = inf16
        for j in range(3):
            pick_v[pl.ds(j * 16, 16)] = ctr_idx

        @plsc.parallel_loop(0, P // 16, unroll=4,
                            carry=jnp.zeros((16,), jnp.int32))
        def scan_body(j, cnt_vec):
            off = j * 16
            pxv = px_v[pl.ds(off, 16)]
            pyv = py_v[pl.ds(off, 16)]
            pzv = pz_v[pl.ds(off, 16)]
            dx = pxv - cxs
            dy = pyv - cys
            dz = pzv - czs
            d2 = (dx * dx + dy * dy) + dz * dz
            m = d2 <= R2
            mi = m.astype(jnp.int32)
            tgt = cnt_vec + plsc.cumsum(mi) - mi
            plsc.store_scatter(cd2_v, [tgt], d2, mask=m)
            plsc.store_scatter(cix_v, [tgt], lane + off, mask=m)
            return cnt_vec + plsc.all_reduce_population_count(m)

        cnt = scan_body[0]

        def run_bisect(_):
            def bis_body(_, lohi):
                lo, hi = lohi
                mid = (lo + hi) * 0.5
                acc = jnp.zeros((16,), jnp.int32)
                for j in range(8):
                    v = cd2_v[pl.ds(j * 16, 16)]
                    acc = acc + plsc.all_reduce_population_count(v <= mid)
                ge = acc >= K
                return (jnp.where(ge, lo, mid), jnp.where(ge, mid, hi))

            lohi0 = (jnp.zeros((16,), jnp.float32),
                     jnp.full((16,), R2, jnp.float32))
            _, hi = lax.fori_loop(0, 26, bis_body, lohi0)
            return hi

        t = lax.cond(cnt > K, run_bisect,
                     lambda _: jnp.full((16,), R2, jnp.float32), operand=None)

        c2_vec = jnp.zeros((16,), jnp.int32)
        for j in range(8):
            v = cd2_v[pl.ds(j * 16, 16)]
            ix = cix_v[pl.ds(j * 16, 16)]
            m = v <= t
            mi = m.astype(jnp.int32)
            tgt = c2_vec + plsc.cumsum(mi) - mi
            plsc.store_scatter(pick_v, [tgt], ix, mask=m)
            c2_vec = c2_vec + plsc.all_reduce_population_count(m)

        rad = jnp.float32(RADIUS)
        lanes4 = lane * 4
        for hk in range(2):
            idx = pick_v[pl.ds(hk * 16, 16)]
            tgt = lanes4 + (i * K + hk * 16) * 4
            plsc.store_scatter(
                rel4_v, [tgt], (plsc.load_gather(px_v, [idx]) - cxs) / rad)
            plsc.store_scatter(
                rel4_v, [tgt + 1], (plsc.load_gather(py_v, [idx]) - cys) / rad)
            plsc.store_scatter(
                rel4_v, [tgt + 2], (plsc.load_gather(pz_v, [idx]) - czs) / rad)
        return 0

    lax.fori_loop(0, CPW, center_body, 0)

    base = wid * CPW
    pltpu.sync_copy(rel4_v, rel4_o.at[pl.ds(base * K * 4, CPW * K * 4)])
    pltpu.sync_copy(fpsb_v, fpsb_o.at[pl.ds(base, CPW)])


def _run_sc(pxyz, batch, fxyz, sel):
    mesh = plsc.VectorSubcoreMesh(core_axis_name="c", subcore_axis_name="s")
    f32 = jnp.float32
    i32 = jnp.int32
    kfn = pl.kernel(
        _sc_body,
        mesh=mesh,
        compiler_params=pltpu.CompilerParams(needs_layout_passes=False),
        out_type=(
            jax.ShapeDtypeStruct((B * S * K * 4,), f32),
            jax.ShapeDtypeStruct((B * S,), i32),
        ),
        scratch_types=[
            pltpu.VMEM((P,), f32),
            pltpu.VMEM((P,), f32),
            pltpu.VMEM((P,), f32),
            pltpu.VMEM((P,), i32),
            pltpu.VMEM((CPW,), f32),
            pltpu.VMEM((CPW,), f32),
            pltpu.VMEM((CPW,), f32),
            pltpu.VMEM((CPW,), i32),
            pltpu.VMEM((CAND,), f32),
            pltpu.VMEM((CAND,), i32),
            pltpu.VMEM((CAND,), i32),
            pltpu.VMEM((CPW * K * 4,), f32),
            pltpu.VMEM((CPW,), i32),
        ],
    )
    return kfn(pxyz, batch, fxyz, sel)


CH = 256
GRID = (B * S) // CH


def _mlp_body(rel4_ref,
              w1_ref, b1_ref, w2_ref, b2_ref, w3_ref, b3_ref,
              wg1_ref, bg1_ref, wg2_ref, bg2_ref, wg3_ref, bg3_ref,
              out_ref):
    h = jnp.dot(rel4_ref[...], w1_ref[...],
                preferred_element_type=jnp.float32) + b1_ref[...]
    h = jnp.maximum(h, 0.0)
    h = jnp.dot(h, w2_ref[...], preferred_element_type=jnp.float32) + b2_ref[...]
    h = jnp.maximum(h, 0.0)
    h = jnp.dot(h, w3_ref[...], preferred_element_type=jnp.float32) + b3_ref[...]
    h = jnp.maximum(h, 0.0)

    pooled = jnp.max(h.reshape(CH, K, 128), axis=1)

    g = jnp.dot(pooled, wg1_ref[...], preferred_element_type=jnp.float32) + bg1_ref[...]
    g = jnp.maximum(g, 0.0)
    g = jnp.dot(g, wg2_ref[...], preferred_element_type=jnp.float32) + bg2_ref[...]
    g = jnp.maximum(g, 0.0)
    g = jnp.dot(g, wg3_ref[...], preferred_element_type=jnp.float32) + bg3_ref[...]
    g = jnp.maximum(g, 0.0)
    out_ref[...] = g


def _run_mlp(rel4, w1p, b1, w2t, b2, w3t, b3,
             wg1t, bg1, wg2t, bg2, wg3t, bg3):
    N = B * S
    full = lambda shape: pl.BlockSpec(shape, lambda g: (0,) * len(shape))
    return pl.pallas_call(
        _mlp_body,
        grid=(GRID,),
        in_specs=[
            pl.BlockSpec((CH * K, 4), lambda g: (g, 0)),
            full((4, 64)), full((1, 64)),
            full((64, 64)), full((1, 64)),
            full((64, 128)), full((1, 128)),
            full((128, 128)), full((1, 128)),
            full((128, 256)), full((1, 256)),
            full((256, 256)), full((1, 256)),
        ],
        out_specs=pl.BlockSpec((CH, 256), lambda g: (g, 0)),
        out_shape=jax.ShapeDtypeStruct((N, 256), jnp.float32),
    )(rel4, w1p, b1, w2t, b2, w3t, b3,
      wg1t, bg1, wg2t, bg2, wg3t, bg3)


def kernel(points, batch, W1, b1, W2, b2, W3, b3, Wg1, bg1, Wg2, bg2, Wg3, bg3):
    pts = points.reshape(B, P, 3)
    pxyz = jnp.concatenate(
        [pts[:, :, 0], pts[:, :, 1], pts[:, :, 2]], axis=0)

    sel, fxyz = _run_fps(pxyz)

    rel4, fpsb = _run_sc(pxyz, batch, fxyz, sel)

    E = B * S * K
    w1p = jnp.concatenate([W1.T, jnp.zeros((1, 64), jnp.float32)], axis=0)

    g = _run_mlp(rel4.reshape(E, 4),
                 w1p, b1.reshape(1, -1), W2.T, b2.reshape(1, -1),
                 W3.T, b3.reshape(1, -1), Wg1.T, bg1.reshape(1, -1),
                 Wg2.T, bg2.reshape(1, -1), Wg3.T, bg3.reshape(1, -1))

    fps_pts = jnp.stack(
        [fxyz[0:B], fxyz[B:2 * B], fxyz[2 * B:3 * B]], axis=-1
    ).reshape(B * S, 3)
    return fps_pts, g, fpsb.reshape(B * S)


# speedup vs baseline: 1.1461x; 1.1461x over previous
import functools

import jax
import jax.numpy as jnp
from jax import lax
from jax.experimental import pallas as pl
from jax.experimental.pallas import tpu as pltpu
from jax.experimental.pallas import tpu_sc as plsc

B, P, S, K = 16, 2048, 256, 32
RADIUS = 0.15
R2 = RADIUS * RADIUS
NEG_INF = float("-inf")


def _fps_body(pxyz_ref, sel_ref, fxyz_ref):
    pxyz = pxyz_ref[...]
    px = pxyz[0:B]
    py = pxyz[B:2 * B]
    pz = pxyz[2 * B:3 * B]
    iota_p3 = lax.broadcasted_iota(jnp.int32, (3 * B, P), 1)
    iota_s = lax.broadcasted_iota(jnp.int32, (B, S), 1)
    iota_s3 = lax.broadcasted_iota(jnp.int32, (3 * B, S), 1)

    l0 = pxyz[:, 0:1]
    sel0 = jnp.zeros((B, S), jnp.int32)
    fxyz0 = jnp.where(iota_s3 == 0, l0, 0.0)
    dist0 = jnp.full((B, P), jnp.inf, jnp.float32)

    def body(i, carry):
        dist, sel, fxyz, l = carry
        lx = l[0:B]
        ly = l[B:2 * B]
        lz = l[2 * B:3 * B]
        dx = px - lx
        dy = py - ly
        dz = pz - lz
        d = (dx * dx + dy * dy) + dz * dz
        dist = jnp.minimum(dist, d)
        nxt = jnp.argmax(dist, axis=1).astype(jnp.int32)[:, None]
        nxt3 = jnp.concatenate([nxt, nxt, nxt], axis=0)
        onehot = iota_p3 == nxt3
        l = jnp.sum(jnp.where(onehot, pxyz, 0.0), axis=1, keepdims=True)
        sel = jnp.where(iota_s == i, nxt, sel)
        fxyz = jnp.where(iota_s3 == i, l, fxyz)
        return (dist, sel, fxyz, l)

    carry = (dist0, sel0, fxyz0, l0)
    _, sel, fxyz, _ = lax.fori_loop(1, S, body, carry)
    sel_ref[...] = sel
    fxyz_ref[...] = fxyz


def _run_fps(pxyz):
    return pl.pallas_call(
        _fps_body,
        out_shape=(
            jax.ShapeDtypeStruct((B, S), jnp.int32),
            jax.ShapeDtypeStruct((3 * B, S), jnp.float32),
        ),
    )(pxyz)


NWORK = 32
CPW = (B * S) // NWORK
CAND = 2064


def _sc_body(pxyz, batch, fxyz, sel,
             relx_o, rely_o, relz_o, fpsb_o,
             px_v, py_v, pz_v, bat_v, cx_v, cy_v, cz_v, sel_v,
             cd2_v, cix_v, pick_v, rx_v, ry_v, rz_v, fpsb_v):
    wid = lax.axis_index("s") * 2 + lax.axis_index("c")
    b = wid // 2
    half = wid % 2

    pltpu.sync_copy(pxyz.at[b], px_v)
    pltpu.sync_copy(pxyz.at[B + b], py_v)
    pltpu.sync_copy(pxyz.at[2 * B + b], pz_v)
    pltpu.sync_copy(batch.at[pl.ds(b * P, P)], bat_v)
    pltpu.sync_copy(fxyz.at[b, pl.ds(half * CPW, CPW)], cx_v)
    pltpu.sync_copy(fxyz.at[B + b, pl.ds(half * CPW, CPW)], cy_v)
    pltpu.sync_copy(fxyz.at[2 * B + b, pl.ds(half * CPW, CPW)], cz_v)
    pltpu.sync_copy(sel.at[b, pl.ds(half * CPW, CPW)], sel_v)

    lane = lax.iota(jnp.int32, 16)

    for j in range(CPW // 16):
        sv = sel_v[pl.ds(j * 16, 16)]
        fpsb_v[pl.ds(j * 16, 16)] = plsc.load_gather(bat_v, [sv])

    inf16 = jnp.full((16,), jnp.inf, jnp.float32)

    def center_body(i, _):
        spl_i = jnp.full((16,), 0, jnp.int32) + i
        cxs = plsc.load_gather(cx_v, [spl_i])
        cys = plsc.load_gather(cy_v, [spl_i])
        czs = plsc.load_gather(cz_v, [spl_i])
        ctr_idx = plsc.load_gather(sel_v, [spl_i])

        for j in range(8):
            cd2_v[pl.ds(j * 16, 16)] = inf16
        for j in range(3):
            pick_v[pl.ds(j * 16, 16)] = ctr_idx

        @plsc.parallel_loop(0, P // 16, unroll=4,
                            carry=jnp.zeros((16,), jnp.int32))
        def scan_body(j, cnt_vec):
            off = j * 16
            pxv = px_v[pl.ds(off, 16)]
            pyv = py_v[pl.ds(off, 16)]
            pzv = pz_v[pl.ds(off, 16)]
            dx = pxv - cxs
            dy = pyv - cys
            dz = pzv - czs
            d2 = (dx * dx + dy * dy) + dz * dz
            m = d2 <= R2
            mi = m.astype(jnp.int32)
            tgt = cnt_vec + plsc.cumsum(mi) - mi
            plsc.store_scatter(cd2_v, [tgt], d2, mask=m)
            plsc.store_scatter(cix_v, [tgt], lane + off, mask=m)
            return cnt_vec + plsc.all_reduce_population_count(m)

        cnt = scan_body[0]

        def run_bisect(_):
            def bis_body(_, lohi):
                lo, hi = lohi
                mid = (lo + hi) * 0.5
                acc = jnp.zeros((16,), jnp.int32)
                for j in range(8):
                    v = cd2_v[pl.ds(j * 16, 16)]
                    acc = acc + plsc.all_reduce_population_count(v <= mid)
                ge = acc >= K
                return (jnp.where(ge, lo, mid), jnp.where(ge, mid, hi))

            lohi0 = (jnp.zeros((16,), jnp.float32),
                     jnp.full((16,), R2, jnp.float32))
            _, hi = lax.fori_loop(0, 26, bis_body, lohi0)
            return hi

        t = lax.cond(cnt > K, run_bisect,
                     lambda _: jnp.full((16,), R2, jnp.float32), operand=None)

        c2_vec = jnp.zeros((16,), jnp.int32)
        for j in range(8):
            v = cd2_v[pl.ds(j * 16, 16)]
            ix = cix_v[pl.ds(j * 16, 16)]
            m = v <= t
            mi = m.astype(jnp.int32)
            tgt = c2_vec + plsc.cumsum(mi) - mi
            plsc.store_scatter(pick_v, [tgt], ix, mask=m)
            c2_vec = c2_vec + plsc.all_reduce_population_count(m)

        rad = jnp.float32(RADIUS)
        for hk in range(2):
            idx = pick_v[pl.ds(hk * 16, 16)]
            off = i * K + hk * 16
            rx_v[pl.ds(off, 16)] = (plsc.load_gather(px_v, [idx]) - cxs) / rad
            ry_v[pl.ds(off, 16)] = (plsc.load_gather(py_v, [idx]) - cys) / rad
            rz_v[pl.ds(off, 16)] = (plsc.load_gather(pz_v, [idx]) - czs) / rad
        return 0

    lax.fori_loop(0, CPW, center_body, 0)

    base = wid * CPW
    pltpu.sync_copy(rx_v, relx_o.at[pl.ds(base * K, CPW * K)])
    pltpu.sync_copy(ry_v, rely_o.at[pl.ds(base * K, CPW * K)])
    pltpu.sync_copy(rz_v, relz_o.at[pl.ds(base * K, CPW * K)])
    pltpu.sync_copy(fpsb_v, fpsb_o.at[pl.ds(base, CPW)])


def _run_sc(pxyz, batch, fxyz, sel):
    mesh = plsc.VectorSubcoreMesh(core_axis_name="c", subcore_axis_name="s")
    f32 = jnp.float32
    i32 = jnp.int32
    kfn = pl.kernel(
        _sc_body,
        mesh=mesh,
        compiler_params=pltpu.CompilerParams(needs_layout_passes=False),
        out_type=(
            jax.ShapeDtypeStruct((B * S * K,), f32),
            jax.ShapeDtypeStruct((B * S * K,), f32),
            jax.ShapeDtypeStruct((B * S * K,), f32),
            jax.ShapeDtypeStruct((B * S,), i32),
        ),
        scratch_types=[
            pltpu.VMEM((P,), f32),
            pltpu.VMEM((P,), f32),
            pltpu.VMEM((P,), f32),
            pltpu.VMEM((P,), i32),
            pltpu.VMEM((CPW,), f32),
            pltpu.VMEM((CPW,), f32),
            pltpu.VMEM((CPW,), f32),
            pltpu.VMEM((CPW,), i32),
            pltpu.VMEM((CAND,), f32),
            pltpu.VMEM((CAND,), i32),
            pltpu.VMEM((CAND,), i32),
            pltpu.VMEM((CPW * K,), f32),
            pltpu.VMEM((CPW * K,), f32),
            pltpu.VMEM((CPW * K,), f32),
            pltpu.VMEM((CPW,), i32),
        ],
    )
    return kfn(pxyz, batch, fxyz, sel)


CH = 256
GRID = (B * S) // CH


def _mlp_body(rel4_ref,
              w1_ref, b1_ref, w2_ref, b2_ref, w3_ref, b3_ref,
              wg1_ref, bg1_ref, wg2_ref, bg2_ref, wg3_ref, bg3_ref,
              out_ref):
    h = jnp.dot(rel4_ref[...], w1_ref[...],
                preferred_element_type=jnp.float32) + b1_ref[...]
    h = jnp.maximum(h, 0.0)
    h = jnp.dot(h, w2_ref[...], preferred_element_type=jnp.float32) + b2_ref[...]
    h = jnp.maximum(h, 0.0)
    h = jnp.dot(h, w3_ref[...], preferred_element_type=jnp.float32) + b3_ref[...]
    h = jnp.maximum(h, 0.0)

    pooled = jnp.max(h.reshape(CH, K, 128), axis=1)

    g = jnp.dot(pooled, wg1_ref[...], preferred_element_type=jnp.float32) + bg1_ref[...]
    g = jnp.maximum(g, 0.0)
    g = jnp.dot(g, wg2_ref[...], preferred_element_type=jnp.float32) + bg2_ref[...]
    g = jnp.maximum(g, 0.0)
    g = jnp.dot(g, wg3_ref[...], preferred_element_type=jnp.float32) + bg3_ref[...]
    g = jnp.maximum(g, 0.0)
    out_ref[...] = g


def _run_mlp(rel4, w1p, b1, w2t, b2, w3t, b3,
             wg1t, bg1, wg2t, bg2, wg3t, bg3):
    N = B * S
    full = lambda shape: pl.BlockSpec(shape, lambda g: (0,) * len(shape))
    return pl.pallas_call(
        _mlp_body,
        grid=(GRID,),
        in_specs=[
            pl.BlockSpec((CH * K, 4), lambda g: (g, 0)),
            full((4, 64)), full((1, 64)),
            full((64, 64)), full((1, 64)),
            full((64, 128)), full((1, 128)),
            full((128, 128)), full((1, 128)),
            full((128, 256)), full((1, 256)),
            full((256, 256)), full((1, 256)),
        ],
        out_specs=pl.BlockSpec((CH, 256), lambda g: (g, 0)),
        out_shape=jax.ShapeDtypeStruct((N, 256), jnp.float32),
    )(rel4, w1p, b1, w2t, b2, w3t, b3,
      wg1t, bg1, wg2t, bg2, wg3t, bg3)


def kernel(points, batch, W1, b1, W2, b2, W3, b3, Wg1, bg1, Wg2, bg2, Wg3, bg3):
    pts = points.reshape(B, P, 3)
    pxyz = jnp.concatenate(
        [pts[:, :, 0], pts[:, :, 1], pts[:, :, 2]], axis=0)

    sel, fxyz = _run_fps(pxyz)

    relx, rely, relz, fpsb = _run_sc(pxyz, batch, fxyz, sel)

    E = B * S * K
    rel4 = jnp.concatenate(
        [relx.reshape(E, 1), rely.reshape(E, 1), relz.reshape(E, 1),
         jnp.zeros((E, 1), jnp.float32)], axis=1)
    w1p = jnp.concatenate([W1.T, jnp.zeros((1, 64), jnp.float32)], axis=0)

    g = _run_mlp(rel4,
                 w1p, b1.reshape(1, -1), W2.T, b2.reshape(1, -1),
                 W3.T, b3.reshape(1, -1), Wg1.T, bg1.reshape(1, -1),
                 Wg2.T, bg2.reshape(1, -1), Wg3.T, bg3.reshape(1, -1))

    fps_pts = jnp.stack(
        [fxyz[0:B], fxyz[B:2 * B], fxyz[2 * B:3 * B]], axis=-1
    ).reshape(B * S, 3)
    return fps_pts, g, fpsb.reshape(B * S)
